# baseline (speedup 1.0000x reference)
import functools

import jax
import jax.numpy as jnp
from jax import lax
from jax.experimental import pallas as pl
from jax.experimental.pallas import tpu as pltpu, tpu_sc as plsc

N_NODES = 10000
N_EDGES = 320000
C = 128
N_GRAPHS = 64
N_CONVS = 12

NC, NS, L = 2, 16, 16
NW = NC * NS
NPAD = 10240
ROWS_PER_TILE = NPAD // NS
BSZ = 128
EPW = 10240
NB = EPW // BSZ
E_PAD = NW * EPW
DW = 16


CH = 40
NCHUNK = NB // CH


def _zero_rows(rows2d):
    def body(i, _):
        for k in range(C // L):
            rows2d[i, pl.ds(k * L, L)] = jnp.zeros((L,), jnp.float32)
        return 0
    lax.fori_loop(0, BSZ, body, 0)


def _segsum_body(h_hbm, src_hbm, dst_hbm, out_hbm, src_c, dst_c, rows_v,
                 acc_sh, sem_g, sem_s0, sem_s1):
    c = lax.axis_index("c")
    s = lax.axis_index("s")
    wid = s * NC + c
    row0 = s * ROWS_PER_TILE

    _zero_rows(rows_v.at[0])
    for k in range(ROWS_PER_TILE // BSZ):
        pltpu.sync_copy(rows_v.at[0], acc_sh.at[pl.ds(row0 + k * BSZ, BSZ)])
    plsc.subcore_barrier()

    sems = (sem_s0, sem_s1)

    def chunk(cc, _):
        base = cc * CH
        pltpu.sync_copy(src_hbm.at[wid, pl.ds(base, CH)], src_c)
        pltpu.sync_copy(dst_hbm.at[wid, pl.ds(base, CH)], dst_c)
        sc = [None, None]
        g = [None, None]
        for t in range(CH):
            b = t % 2
            if sc[b] is not None:
                sc[b].wait()
            g[b] = pltpu.async_copy(h_hbm.at[src_c.at[t]], rows_v.at[b],
                                    sem_g)
            if t > 0:
                g[1 - b].wait()
                sc[1 - b] = pltpu.async_copy(rows_v.at[1 - b],
                                             acc_sh.at[dst_c.at[t - 1]],
                                             sems[1 - b], add=True)
        bl = (CH - 1) % 2
        g[bl].wait()
        sc[bl] = pltpu.async_copy(rows_v.at[bl], acc_sh.at[dst_c.at[CH - 1]],
                                  sems[bl], add=True)
        for b in range(2):
            sc[b].wait()
        return 0
    lax.fori_loop(0, NCHUNK, chunk, 0)
    plsc.subcore_barrier()

    for k in range(ROWS_PER_TILE // BSZ):
        r = row0 + k * BSZ
        pltpu.sync_copy(acc_sh.at[pl.ds(r, BSZ)], rows_v.at[0])
        pltpu.sync_copy(rows_v.at[0], out_hbm.at[c, pl.ds(r, BSZ)])


def _deg_body(dst_hbm, out_hbm, dst_c, rows_v, acc_sh, sem_s0, sem_s1):
    c = lax.axis_index("c")
    s = lax.axis_index("s")
    wid = s * NC + c
    row0 = s * ROWS_PER_TILE

    _zero_rows(rows_v.at[0])
    for k in range(ROWS_PER_TILE // BSZ):
        pltpu.sync_copy(rows_v.at[0], acc_sh.at[pl.ds(row0 + k * BSZ, BSZ)])

    def fill(i, _):
        for k in range(C // L):
            rows_v[0, i, pl.ds(k * L, L)] = jnp.ones((L,), jnp.float32)
        return 0
    lax.fori_loop(0, BSZ, fill, 0)
    plsc.subcore_barrier()

    sems = (sem_s0, sem_s1)

    def chunk(cc, _):
        base = cc * CH
        pltpu.sync_copy(dst_hbm.at[wid, pl.ds(base, CH)], dst_c)
        sc = [None, None]
        for t in range(CH):
            b = t % 2
            if sc[b] is not None:
                sc[b].wait()
            sc[b] = pltpu.async_copy(rows_v.at[0], acc_sh.at[dst_c.at[t]],
                                     sems[b], add=True)
        for b in range(2):
            sc[b].wait()
        return 0
    lax.fori_loop(0, NCHUNK, chunk, 0)
    plsc.subcore_barrier()

    for k in range(ROWS_PER_TILE // BSZ):
        r = row0 + k * BSZ
        pltpu.sync_copy(acc_sh.at[pl.ds(r, BSZ)], rows_v.at[0])
        pltpu.sync_copy(rows_v.at[0], out_hbm.at[c, pl.ds(r, BSZ)])


@functools.cache
def _get_sc_kernels():
    mesh = plsc.VectorSubcoreMesh(core_axis_name="c", subcore_axis_name="s",
                                  num_cores=NC, num_subcores=NS)

    segsum = pl.kernel(
        _segsum_body,
        out_type=jax.ShapeDtypeStruct((NC, NPAD, C), jnp.float32),
        mesh=mesh,
        scratch_types=[
            pltpu.VMEM((CH, BSZ), jnp.int32),
            pltpu.VMEM((CH, BSZ), jnp.int32),
            pltpu.VMEM((2, BSZ, C), jnp.float32),
            pltpu.VMEM_SHARED((NPAD, C), jnp.float32),
            pltpu.SemaphoreType.DMA,
            pltpu.SemaphoreType.DMA,
            pltpu.SemaphoreType.DMA,
        ],
    )

    degree = pl.kernel(
        _deg_body,
        out_type=jax.ShapeDtypeStruct((NC, NPAD, C), jnp.float32),
        mesh=mesh,
        scratch_types=[
            pltpu.VMEM((CH, BSZ), jnp.int32),
            pltpu.VMEM((1, BSZ, C), jnp.float32),
            pltpu.VMEM_SHARED((NPAD, C), jnp.float32),
            pltpu.SemaphoreType.DMA,
            pltpu.SemaphoreType.DMA,
        ],
    )

    return segsum, degree


ROW_BLK = 1000
N_BLKS = N_NODES // ROW_BLK


def _layer_body(act, p_ref, deg_ref, h_ref, wl_ref, wr_ref, b_ref, o_ref):
    psum = p_ref[0] + p_ref[1]
    deg = deg_ref[0, :, 0:1] + deg_ref[1, :, 0:1]
    mean = psum / jnp.maximum(deg, 1.0)
    out = (jnp.dot(mean, wl_ref[...], preferred_element_type=jnp.float32)
           + jnp.dot(h_ref[...], wr_ref[...], preferred_element_type=jnp.float32)
           + b_ref[...])
    if act == "relu":
        out = jnp.maximum(out, 0.0)
    elif act == "leaky":
        out = jnp.where(out >= 0.0, out, 0.01 * out)
    o_ref[...] = out


@functools.cache
def _layer_fn(act):
    return pl.pallas_call(
        functools.partial(_layer_body, act),
        grid=(N_BLKS,),
        in_specs=[
            pl.BlockSpec((NC, ROW_BLK, C), lambda j: (0, j, 0)),
            pl.BlockSpec((NC, ROW_BLK, DW), lambda j: (0, j, 0)),
            pl.BlockSpec((ROW_BLK, C), lambda j: (j, 0)),
            pl.BlockSpec((C, C), lambda j: (0, 0)),
            pl.BlockSpec((C, C), lambda j: (0, 0)),
            pl.BlockSpec((1, C), lambda j: (0, 0)),
        ],
        out_specs=pl.BlockSpec((ROW_BLK, C), lambda j: (j, 0)),
        out_shape=jax.ShapeDtypeStruct((N_NODES, C), jnp.float32),
    )


def _pool_body(h_ref, bidx_ref, hw_ref, hb_ref, o_ref, acc_s, acc_c):
    j = pl.program_id(0)
    iota = lax.broadcasted_iota(jnp.int32, (N_GRAPHS, ROW_BLK), 0)
    pt = (iota == bidx_ref[0]).astype(jnp.float32)
    ps = jnp.dot(pt, h_ref[...], preferred_element_type=jnp.float32)
    pc = jnp.dot(pt, jnp.ones((ROW_BLK, C), jnp.float32),
                 preferred_element_type=jnp.float32)

    @pl.when(j == 0)
    def _():
        acc_s[...] = ps
        acc_c[...] = pc

    @pl.when(j > 0)
    def _():
        acc_s[...] += ps
        acc_c[...] += pc

    @pl.when(j == N_BLKS - 1)
    def _():
        pooled = acc_s[...] / jnp.maximum(acc_c[...], 1.0)
        o_ref[...] = (jnp.dot(pooled, hw_ref[...],
                              preferred_element_type=jnp.float32)
                      + hb_ref[...])


_pool_fn = pl.pallas_call(
    _pool_body,
    grid=(N_BLKS,),
    in_specs=[
        pl.BlockSpec((ROW_BLK, C), lambda j: (j, 0)),
        pl.BlockSpec((1, 1, ROW_BLK), lambda j: (j, 0, 0)),
        pl.BlockSpec((C, C), lambda j: (0, 0)),
        pl.BlockSpec((1, C), lambda j: (0, 0)),
    ],
    out_specs=pl.BlockSpec((N_GRAPHS, C), lambda j: (0, 0)),
    out_shape=jax.ShapeDtypeStruct((N_GRAPHS, C), jnp.float32),
    scratch_shapes=[
        pltpu.VMEM((N_GRAPHS, C), jnp.float32),
        pltpu.VMEM((N_GRAPHS, C), jnp.float32),
    ],
)

_ACTS = ["relu", "relu", "relu", "leaky",
         "relu", "relu", "relu", "leaky",
         "relu", "relu", "relu", "none"]


def kernel(x, Wl, Wr, b, head_W, head_b, edge_index, batch_idx):
    src = edge_index[0].astype(jnp.int32)
    dst = edge_index[1].astype(jnp.int32)
    pad = E_PAD - N_EDGES
    src_p = jnp.concatenate([src, jnp.zeros((pad,), jnp.int32)])
    dst_p = jnp.concatenate([dst, jnp.full((pad,), NPAD - 1, jnp.int32)])
    src3 = src_p.reshape(NW, NB, BSZ)
    dst3 = dst_p.reshape(NW, NB, BSZ)

    WlT = jnp.transpose(Wl, (0, 2, 1))
    WrT = jnp.transpose(Wr, (0, 2, 1))
    b2 = b.reshape(N_CONVS, 1, C)
    hwT = jnp.zeros((C, C), jnp.float32).at[:, :head_W.shape[0]].set(head_W.T)
    hb2 = jnp.zeros((1, C), jnp.float32).at[0, :head_b.shape[0]].set(head_b)

    segsum, degree = _get_sc_kernels()
    deg = degree(dst3)[:, :, :DW]
    h = x
    for i in range(N_CONVS):
        part = segsum(h, src3, dst3)
        h = _layer_fn(_ACTS[i])(part, deg, h, WlT[i], WrT[i], b2[i])

    bidx3 = batch_idx.astype(jnp.int32).reshape(N_BLKS, 1, ROW_BLK)
    pooled = _pool_fn(h, bidx3, hwT, hb2)
    return pooled[:, :head_W.shape[0]]


# speedup vs baseline: 1.1318x; 1.1318x over previous
import functools

import jax
import jax.numpy as jnp
from jax import lax
from jax.experimental import pallas as pl
from jax.experimental.pallas import tpu as pltpu, tpu_sc as plsc

N_NODES = 10000
N_EDGES = 320000
C = 128
N_GRAPHS = 64
N_CONVS = 12

NC, NS, L = 2, 16, 16
NW = NC * NS
NPAD = 10240
ROWS_PER_TILE = NPAD // NS
BSZ = 128
EPW = 10240
NB = EPW // BSZ
E_PAD = NW * EPW
DW = 16


CH = 40
NCHUNK = NB // CH


def _zero_rows(rows2d):
    def body(i, _):
        for k in range(C // L):
            rows2d[i, pl.ds(k * L, L)] = jnp.zeros((L,), jnp.float32)
        return 0
    lax.fori_loop(0, BSZ, body, 0)


def _segsum_body(h_hbm, src_hbm, dst_hbm, out_hbm, src_c, dst_c, rows_v,
                 acc_sh, sem_g, sem_s0, sem_s1):
    c = lax.axis_index("c")
    s = lax.axis_index("s")
    wid = s * NC + c
    row0 = s * ROWS_PER_TILE

    _zero_rows(rows_v.at[0])
    for k in range(ROWS_PER_TILE // BSZ):
        pltpu.sync_copy(rows_v.at[0], acc_sh.at[pl.ds(row0 + k * BSZ, BSZ)])
    plsc.subcore_barrier()

    sems = (sem_s0, sem_s1)

    def chunk(cc, _):
        base = cc * CH
        pltpu.sync_copy(src_hbm.at[wid, pl.ds(base, CH)], src_c)
        pltpu.sync_copy(dst_hbm.at[wid, pl.ds(base, CH)], dst_c)
        sc = [None, None]
        g = [None, None]
        for t in range(CH):
            b = t % 2
            if sc[b] is not None:
                sc[b].wait()
            g[b] = pltpu.async_copy(h_hbm.at[src_c.at[t]], rows_v.at[b],
                                    sem_g)
            if t > 0:
                g[1 - b].wait()
                sc[1 - b] = pltpu.async_copy(rows_v.at[1 - b],
                                             acc_sh.at[dst_c.at[t - 1]],
                                             sems[1 - b], add=True)
        bl = (CH - 1) % 2
        g[bl].wait()
        sc[bl] = pltpu.async_copy(rows_v.at[bl], acc_sh.at[dst_c.at[CH - 1]],
                                  sems[bl], add=True)
        for b in range(2):
            sc[b].wait()
        return 0
    lax.fori_loop(0, NCHUNK, chunk, 0)
    plsc.subcore_barrier()

    for k in range(ROWS_PER_TILE // BSZ):
        r = row0 + k * BSZ
        pltpu.sync_copy(acc_sh.at[pl.ds(r, BSZ)], rows_v.at[0])
        pltpu.sync_copy(rows_v.at[0], out_hbm.at[c, pl.ds(r, BSZ)])


@functools.cache
def _get_sc_kernels():
    mesh = plsc.VectorSubcoreMesh(core_axis_name="c", subcore_axis_name="s",
                                  num_cores=NC, num_subcores=NS)

    segsum = pl.kernel(
        _segsum_body,
        out_type=jax.ShapeDtypeStruct((NC, NPAD, C), jnp.float32),
        mesh=mesh,
        scratch_types=[
            pltpu.VMEM((CH, BSZ), jnp.int32),
            pltpu.VMEM((CH, BSZ), jnp.int32),
            pltpu.VMEM((2, BSZ, C), jnp.float32),
            pltpu.VMEM_SHARED((NPAD, C), jnp.float32),
            pltpu.SemaphoreType.DMA,
            pltpu.SemaphoreType.DMA,
            pltpu.SemaphoreType.DMA,
        ],
    )

    return segsum


ROW_BLK = 1000
N_BLKS = N_NODES // ROW_BLK


def _layer_body(act, p_ref, deg_ref, h_ref, wl_ref, wr_ref, b_ref, o_ref):
    psum = p_ref[0] + p_ref[1]
    deg = deg_ref[0, :, 0:1] + deg_ref[1, :, 0:1]
    mean = psum / jnp.maximum(deg, 1.0)
    out = (jnp.dot(mean, wl_ref[...], preferred_element_type=jnp.float32)
           + jnp.dot(h_ref[...], wr_ref[...], preferred_element_type=jnp.float32)
           + b_ref[...])
    if act == "relu":
        out = jnp.maximum(out, 0.0)
    elif act == "leaky":
        out = jnp.where(out >= 0.0, out, 0.01 * out)
    o_ref[...] = out


@functools.cache
def _layer_fn(act):
    return pl.pallas_call(
        functools.partial(_layer_body, act),
        grid=(N_BLKS,),
        in_specs=[
            pl.BlockSpec((NC, ROW_BLK, C), lambda j: (0, j, 0)),
            pl.BlockSpec((NC, ROW_BLK, DW), lambda j: (0, j, 0)),
            pl.BlockSpec((ROW_BLK, C), lambda j: (j, 0)),
            pl.BlockSpec((C, C), lambda j: (0, 0)),
            pl.BlockSpec((C, C), lambda j: (0, 0)),
            pl.BlockSpec((1, C), lambda j: (0, 0)),
        ],
        out_specs=pl.BlockSpec((ROW_BLK, C), lambda j: (j, 0)),
        out_shape=jax.ShapeDtypeStruct((N_NODES, C), jnp.float32),
    )


def _pool_body(h_ref, bidx_ref, hw_ref, hb_ref, o_ref, acc_s, acc_c):
    j = pl.program_id(0)
    iota = lax.broadcasted_iota(jnp.int32, (N_GRAPHS, ROW_BLK), 0)
    pt = (iota == bidx_ref[0]).astype(jnp.float32)
    ps = jnp.dot(pt, h_ref[...], preferred_element_type=jnp.float32)
    pc = jnp.dot(pt, jnp.ones((ROW_BLK, C), jnp.float32),
                 preferred_element_type=jnp.float32)

    @pl.when(j == 0)
    def _():
        acc_s[...] = ps
        acc_c[...] = pc

    @pl.when(j > 0)
    def _():
        acc_s[...] += ps
        acc_c[...] += pc

    @pl.when(j == N_BLKS - 1)
    def _():
        pooled = acc_s[...] / jnp.maximum(acc_c[...], 1.0)
        o_ref[...] = (jnp.dot(pooled, hw_ref[...],
                              preferred_element_type=jnp.float32)
                      + hb_ref[...])


_pool_fn = pl.pallas_call(
    _pool_body,
    grid=(N_BLKS,),
    in_specs=[
        pl.BlockSpec((ROW_BLK, C), lambda j: (j, 0)),
        pl.BlockSpec((1, 1, ROW_BLK), lambda j: (j, 0, 0)),
        pl.BlockSpec((C, C), lambda j: (0, 0)),
        pl.BlockSpec((1, C), lambda j: (0, 0)),
    ],
    out_specs=pl.BlockSpec((N_GRAPHS, C), lambda j: (0, 0)),
    out_shape=jax.ShapeDtypeStruct((N_GRAPHS, C), jnp.float32),
    scratch_shapes=[
        pltpu.VMEM((N_GRAPHS, C), jnp.float32),
        pltpu.VMEM((N_GRAPHS, C), jnp.float32),
    ],
)

_ACTS = ["relu", "relu", "relu", "leaky",
         "relu", "relu", "relu", "leaky",
         "relu", "relu", "relu", "none"]


def kernel(x, Wl, Wr, b, head_W, head_b, edge_index, batch_idx):
    src = edge_index[0].astype(jnp.int32)
    dst = edge_index[1].astype(jnp.int32)
    pad = E_PAD - N_EDGES
    src_p = jnp.concatenate([src, jnp.zeros((pad,), jnp.int32)])
    dst_p = jnp.concatenate([dst, jnp.full((pad,), NPAD - 1, jnp.int32)])
    src3 = src_p.reshape(NW, NB, BSZ)
    dst3 = dst_p.reshape(NW, NB, BSZ)

    WlT = jnp.transpose(Wl, (0, 2, 1))
    WrT = jnp.transpose(Wr, (0, 2, 1))
    b2 = b.reshape(N_CONVS, 1, C)
    hwT = jnp.zeros((C, C), jnp.float32).at[:, :head_W.shape[0]].set(head_W.T)
    hb2 = jnp.zeros((1, C), jnp.float32).at[0, :head_b.shape[0]].set(head_b)

    segsum = _get_sc_kernels()
    ones = jnp.ones((N_NODES, C), jnp.float32)
    deg = segsum(ones, src3, dst3)[:, :, :DW]
    h = x
    for i in range(N_CONVS):
        part = segsum(h, src3, dst3)
        h = _layer_fn(_ACTS[i])(part, deg, h, WlT[i], WrT[i], b2[i])

    bidx3 = batch_idx.astype(jnp.int32).reshape(N_BLKS, 1, ROW_BLK)
    pooled = _pool_fn(h, bidx3, hwT, hb2)
    return pooled[:, :head_W.shape[0]]
